# baseline (speedup 1.0000x reference)
import functools

import jax
import jax.numpy as jnp
from jax import lax
from jax.experimental import pallas as pl
from jax.experimental.pallas import tpu as pltpu
from jax.experimental.pallas import tpu_sc as plsc

N_NODES = 100000
N_GRAPHS = 512
D_OUT = 36
CHUNK = 512
SUB = 128
NW = 32
N_CHUNKS = -(-N_NODES // CHUNK)
MAX_ITERS = -(-N_CHUNKS // NW)


def _table_body(ep_ref, out_ref):
    out_ref[...] = jnp.zeros((N_GRAPHS, D_OUT), jnp.float32)
    ep = ep_ref[...]
    for w in range(4):
        out_ref[:, 9 * w:9 * w + 4] = ep


def _build_table(ep):
    return pl.pallas_call(
        _table_body,
        out_shape=jax.ShapeDtypeStruct((N_GRAPHS, D_OUT), jnp.float32),
    )(ep)


def _gather_body(batch_hbm, table_hbm, out_hbm, idx_v, rows_v, sem):
    wid = lax.axis_index("s") * 2 + lax.axis_index("c")
    for i in range(MAX_ITERS):
        c = jnp.minimum(wid + NW * i, N_CHUNKS - 1)
        base = jnp.minimum(c * CHUNK, N_NODES - CHUNK)
        for j in range(CHUNK // SUB):
            pltpu.sync_copy(batch_hbm.at[pl.ds(base + j * SUB, SUB)],
                            idx_v.at[j])
        copies = []
        for j in range(CHUNK // SUB):
            copies.append(pltpu.async_copy(
                table_hbm.at[idx_v.at[j]],
                rows_v.at[pl.ds(j * SUB, SUB), :], sem))
        for cp in copies:
            cp.wait()
        pltpu.sync_copy(rows_v, out_hbm.at[pl.ds(base, CHUNK), :])


@functools.partial(jax.jit, static_argnames=())
def _gather(batch, table):
    mesh = plsc.VectorSubcoreMesh(core_axis_name="c", subcore_axis_name="s")
    return pl.kernel(
        _gather_body,
        out_type=jax.ShapeDtypeStruct((N_NODES, D_OUT), jnp.float32),
        mesh=mesh,
        scratch_types=[
            pltpu.VMEM((CHUNK // SUB, SUB), jnp.int32),
            pltpu.VMEM((CHUNK, D_OUT), jnp.float32),
            pltpu.SemaphoreType.DMA,
        ],
        compiler_params=pltpu.CompilerParams(use_tc_tiling_on_sc=False),
    )(batch, table)


def kernel(batch, positions, external_potential):
    table = _build_table(external_potential.astype(jnp.float32))
    return _gather(batch.astype(jnp.int32), table)


# speedup vs baseline: 1.5593x; 1.5593x over previous
import jax
import jax.numpy as jnp
from jax import lax
from jax.experimental import pallas as pl
from jax.experimental.pallas import tpu as pltpu
from jax.experimental.pallas import tpu_sc as plsc

N_NODES = 100000
N_GRAPHS = 512
D_OUT = 36
SUB = 128
NW = 32
CHUNK_SUBS = 13
CHUNK = CHUNK_SUBS * SUB
CHUNKS_PER_W = 2
N_PAD = NW * CHUNKS_PER_W * CHUNK
PAD_ROWS = N_PAD // SUB


def _table_body(ep_ref, out_ref):
    out_ref[...] = jnp.zeros((N_GRAPHS, D_OUT), jnp.float32)
    ep = ep_ref[...]
    for w in range(4):
        out_ref[:, 9 * w:9 * w + 4] = ep


def _build_table(ep):
    return pl.pallas_call(
        _table_body,
        out_shape=jax.ShapeDtypeStruct((N_GRAPHS, D_OUT), jnp.float32),
    )(ep)


def _gather_body(batch2d_hbm, table_hbm, out_hbm, idx_v, rows_v, sem):
    wid = lax.axis_index("s") * 2 + lax.axis_index("c")
    for k in range(CHUNKS_PER_W):
        row0 = wid * (CHUNKS_PER_W * CHUNK_SUBS) + k * CHUNK_SUBS
        pltpu.sync_copy(batch2d_hbm.at[pl.ds(row0, CHUNK_SUBS)], idx_v)
        for g0 in range(0, CHUNK_SUBS, 4):
            copies = []
            for j in range(g0, min(g0 + 4, CHUNK_SUBS)):
                copies.append(pltpu.async_copy(
                    table_hbm.at[idx_v.at[j]],
                    rows_v.at[j], sem))
            for cp in copies:
                cp.wait()
        pltpu.sync_copy(rows_v, out_hbm.at[pl.ds(row0, CHUNK_SUBS)])


@jax.jit
def _gather(batch2d, table):
    mesh = plsc.VectorSubcoreMesh(core_axis_name="c", subcore_axis_name="s")
    return pl.kernel(
        _gather_body,
        out_type=jax.ShapeDtypeStruct((PAD_ROWS, SUB, D_OUT), jnp.float32),
        mesh=mesh,
        scratch_types=[
            pltpu.VMEM((CHUNK_SUBS, SUB), jnp.int32),
            pltpu.VMEM((CHUNK_SUBS, SUB, D_OUT), jnp.float32),
            pltpu.SemaphoreType.DMA,
        ],
        compiler_params=pltpu.CompilerParams(use_tc_tiling_on_sc=False),
    )(batch2d, table)


def kernel(batch, positions, external_potential):
    table = _build_table(external_potential.astype(jnp.float32))
    batch2d = jnp.pad(batch.astype(jnp.int32), (0, N_PAD - N_NODES)
                      ).reshape(PAD_ROWS, SUB)
    outp = _gather(batch2d, table)
    return outp.reshape(N_PAD, D_OUT)[:N_NODES]


# speedup vs baseline: 2.2280x; 1.4289x over previous
import jax
import jax.numpy as jnp
from jax import lax
from jax.experimental import pallas as pl
from jax.experimental.pallas import tpu as pltpu
from jax.experimental.pallas import tpu_sc as plsc

N_NODES = 100000
N_GRAPHS = 512
D_OUT = 36
SUB = 128
NW = 32
CHUNK_SUBS = 13
CHUNK = CHUNK_SUBS * SUB
N_FULL = N_NODES // CHUNK
N_CHUNKS = N_FULL + 1
MAX_ITERS = -(-N_CHUNKS // NW)
B2D_ROWS = 782
TAIL_IDX_ROW = B2D_ROWS - CHUNK_SUBS
TAIL_SKIP = N_FULL * CHUNK - TAIL_IDX_ROW * SUB
TAIL_ROWS = N_NODES - N_FULL * CHUNK


def _table_body(ep_ref, out_ref):
    out_ref[...] = jnp.zeros((N_GRAPHS, D_OUT), jnp.float32)
    ep = ep_ref[...]
    for w in range(4):
        out_ref[:, 9 * w:9 * w + 4] = ep


def _build_table(ep):
    return pl.pallas_call(
        _table_body,
        out_shape=jax.ShapeDtypeStruct((N_GRAPHS, D_OUT), jnp.float32),
    )(ep)


def _gather_body(batch2d_hbm, table_hbm, out_hbm, idx_v, rows_v, sem):
    wid = lax.axis_index("s") * 2 + lax.axis_index("c")

    def run_chunk(idx_row):
        pltpu.sync_copy(batch2d_hbm.at[pl.ds(idx_row, CHUNK_SUBS)], idx_v)
        for g0 in range(0, CHUNK_SUBS, 4):
            copies = []
            for j in range(g0, min(g0 + 4, CHUNK_SUBS)):
                copies.append(pltpu.async_copy(
                    table_hbm.at[idx_v.at[j]],
                    rows_v.at[pl.ds(j * SUB, SUB), :], sem))
            for cp in copies:
                cp.wait()

    for k in range(MAX_ITERS):
        c = wid + NW * k

        @pl.when(c < N_FULL)
        def _():
            run_chunk(c * CHUNK_SUBS)
            pltpu.sync_copy(rows_v, out_hbm.at[pl.ds(c * CHUNK, CHUNK), :])

        @pl.when(c == N_FULL)
        def _():
            run_chunk(TAIL_IDX_ROW)
            pltpu.sync_copy(
                rows_v.at[pl.ds(TAIL_SKIP, TAIL_ROWS), :],
                out_hbm.at[pl.ds(N_FULL * CHUNK, TAIL_ROWS), :])


@jax.jit
def _gather(batch2d, table):
    mesh = plsc.VectorSubcoreMesh(core_axis_name="c", subcore_axis_name="s")
    return pl.kernel(
        _gather_body,
        out_type=jax.ShapeDtypeStruct((N_NODES, D_OUT), jnp.float32),
        mesh=mesh,
        scratch_types=[
            pltpu.VMEM((CHUNK_SUBS, SUB), jnp.int32),
            pltpu.VMEM((CHUNK, D_OUT), jnp.float32),
            pltpu.SemaphoreType.DMA,
        ],
        compiler_params=pltpu.CompilerParams(use_tc_tiling_on_sc=False),
    )(batch2d, table)


def kernel(batch, positions, external_potential):
    table = _build_table(external_potential.astype(jnp.float32))
    batch2d = jnp.pad(batch.astype(jnp.int32),
                      (0, B2D_ROWS * SUB - N_NODES)).reshape(B2D_ROWS, SUB)
    return _gather(batch2d, table)
